# baseline (speedup 1.0000x reference)
import functools

import jax
import jax.numpy as jnp
from jax import lax
from jax.experimental import pallas as pl
from jax.experimental.pallas import tpu as pltpu
from jax.experimental.pallas import tpu_sc as plsc

_BITS = 8
_LEVELS = float(2 ** _BITS - 1)
_HALF = float(2 ** (_BITS - 1))

_NC, _NS = 2, 16
_NW = _NC * _NS


def _make_sc_copy(n):
    per_w = n // _NW
    nchunks = 8
    ch = per_w // nchunks
    mesh = plsc.VectorSubcoreMesh(
        core_axis_name="c", subcore_axis_name="s",
        num_cores=_NC, num_subcores=_NS,
    )

    @functools.partial(
        pl.kernel,
        out_type=jax.ShapeDtypeStruct((n,), jnp.float32),
        mesh=mesh,
        scratch_types=[pltpu.SemaphoreType.DMA],
    )
    def body(x_hbm, out_hbm, sem):
        wid = lax.axis_index("s") * _NC + lax.axis_index("c")
        base = wid * per_w
        handles = []
        for k in range(nchunks):
            handles.append(pltpu.async_copy(
                x_hbm.at[pl.ds(base + k * ch, ch)],
                out_hbm.at[pl.ds(base + k * ch, ch)],
                sem,
            ))
        for h in handles:
            h.wait()

    return body


def _tc_body(x_ref, q_ref, ms_ref, inv_ref):
    p = pl.program_id(0)
    j = pl.program_id(1)

    @pl.when(p == 0)
    def _phase_minmax():
        bmn = jnp.min(x_ref[...])
        bmx = jnp.max(x_ref[...])

        @pl.when(j == 0)
        def _init():
            ms_ref[0] = bmn
            ms_ref[1] = bmx

        @pl.when(j != 0)
        def _acc():
            ms_ref[0] = jnp.minimum(ms_ref[0], bmn)
            ms_ref[1] = jnp.maximum(ms_ref[1], bmx)

    @pl.when(p == 1)
    def _phase_quant():
        @pl.when(j == 0)
        def _finalize():
            step = (ms_ref[1] - ms_ref[0]) / _LEVELS
            ms_ref[1] = step
            inv_ref[0] = 1.0 / step

        q_ref[...] = jnp.round(
            (x_ref[...] - ms_ref[0]) * inv_ref[0] - _HALF
        ).astype(jnp.int8)


def kernel(input):
    shape = input.shape
    C = shape[-1]
    R = 1
    for s in shape[:-1]:
        R *= s
    x = input.reshape(R, C)

    xc = _make_sc_copy(R * C)(x.reshape(-1))

    nb = 16
    bs = R // nb
    q, ms = pl.pallas_call(
        _tc_body,
        grid=(2, nb),
        in_specs=[pl.BlockSpec((bs, C), lambda p, j: (j, 0))],
        out_specs=[
            pl.BlockSpec((bs, C), lambda p, j: (jnp.where(p == 0, 0, j), 0)),
            pl.BlockSpec(memory_space=pltpu.SMEM),
        ],
        out_shape=[
            jax.ShapeDtypeStruct((R, C), jnp.int8),
            jax.ShapeDtypeStruct((2,), jnp.float32),
        ],
        scratch_shapes=[pltpu.SMEM((1,), jnp.float32)],
        compiler_params=pltpu.CompilerParams(
            dimension_semantics=("arbitrary", "arbitrary"),
        ),
    )(x)

    return (xc.reshape(shape), q.reshape(shape), ms)


# speedup vs baseline: 10.0822x; 10.0822x over previous
import functools

import jax
import jax.numpy as jnp
from jax import lax
from jax.experimental import pallas as pl
from jax.experimental.pallas import tpu as pltpu
from jax.experimental.pallas import tpu_sc as plsc

_BITS = 8
_LEVELS = float(2 ** _BITS - 1)
_HALF = float(2 ** (_BITS - 1))

_NC, _NS = 2, 16
_NW = _NC * _NS


_SC_CH = 65536


def _make_sc_copy(n):
    per_w = n // _NW
    nchunks = per_w // _SC_CH
    mesh = plsc.VectorSubcoreMesh(
        core_axis_name="c", subcore_axis_name="s",
        num_cores=_NC, num_subcores=_NS,
    )

    @functools.partial(
        pl.kernel,
        out_type=jax.ShapeDtypeStruct((n,), jnp.float32),
        mesh=mesh,
        scratch_types=[
            pltpu.VMEM((2, _SC_CH), jnp.float32),
            pltpu.SemaphoreType.DMA((2,)),
            pltpu.SemaphoreType.DMA((2,)),
        ],
    )
    def body(x_hbm, out_hbm, buf, in_sems, out_sems):
        wid = lax.axis_index("s") * _NC + lax.axis_index("c")
        base = wid * per_w

        def _in(k, b):
            return pltpu.async_copy(
                x_hbm.at[pl.ds(base + k * _SC_CH, _SC_CH)],
                buf.at[b], in_sems.at[b],
            )

        def _out(k, b):
            return pltpu.async_copy(
                buf.at[b],
                out_hbm.at[pl.ds(base + k * _SC_CH, _SC_CH)],
                out_sems.at[b],
            )

        h_in = [None, None]
        h_out = [None, None]
        h_in[0] = _in(0, 0)
        for k in range(nchunks):
            b = k % 2
            h_in[b].wait()
            h_out[b] = _out(k, b)
            if k + 1 < nchunks:
                nb_ = (k + 1) % 2
                if h_out[nb_] is not None:
                    h_out[nb_].wait()
                h_in[nb_] = _in(k + 1, nb_)
        h_out[0].wait()
        h_out[1].wait()

    return body


def _tc_body(x_ref, q_ref, ms_ref, inv_ref):
    p = pl.program_id(0)
    j = pl.program_id(1)

    @pl.when(p == 0)
    def _phase_minmax():
        bmn = jnp.min(x_ref[...])
        bmx = jnp.max(x_ref[...])

        @pl.when(j == 0)
        def _init():
            ms_ref[0] = bmn
            ms_ref[1] = bmx

        @pl.when(j != 0)
        def _acc():
            ms_ref[0] = jnp.minimum(ms_ref[0], bmn)
            ms_ref[1] = jnp.maximum(ms_ref[1], bmx)

    @pl.when(p == 1)
    def _phase_quant():
        @pl.when(j == 0)
        def _finalize():
            step = (ms_ref[1] - ms_ref[0]) / _LEVELS
            ms_ref[1] = step
            inv_ref[0] = 1.0 / step

        q_ref[...] = jnp.round(
            (x_ref[...] - ms_ref[0]) * inv_ref[0] - _HALF
        ).astype(jnp.int8)


def kernel(input):
    shape = input.shape
    C = shape[-1]
    R = 1
    for s in shape[:-1]:
        R *= s
    x = input.reshape(R, C)

    xc = _make_sc_copy(R * C)(x.reshape(-1))

    nb = 16
    bs = R // nb
    q, ms = pl.pallas_call(
        _tc_body,
        grid=(2, nb),
        in_specs=[pl.BlockSpec((bs, C), lambda p, j: (j, 0))],
        out_specs=[
            pl.BlockSpec((bs, C), lambda p, j: (jnp.where(p == 0, 0, j), 0)),
            pl.BlockSpec(memory_space=pltpu.SMEM),
        ],
        out_shape=[
            jax.ShapeDtypeStruct((R, C), jnp.int8),
            jax.ShapeDtypeStruct((2,), jnp.float32),
        ],
        scratch_shapes=[pltpu.SMEM((1,), jnp.float32)],
        compiler_params=pltpu.CompilerParams(
            dimension_semantics=("arbitrary", "arbitrary"),
        ),
    )(x)

    return (xc.reshape(shape), q.reshape(shape), ms)


# speedup vs baseline: 21.6607x; 2.1484x over previous
import functools

import jax
import jax.numpy as jnp
from jax import lax
from jax.experimental import pallas as pl
from jax.experimental.pallas import tpu as pltpu
from jax.experimental.pallas import tpu_sc as plsc

_BITS = 8
_LEVELS = float(2 ** _BITS - 1)
_HALF = float(2 ** (_BITS - 1))

_NC, _NS = 2, 16
_NW = _NC * _NS


_SC_CHROWS = 8
_SC_NBUF = 4


def _make_sc_copy(rows, cols):
    rows_per_w = rows // _NW
    nchunks = rows_per_w // _SC_CHROWS
    mesh = plsc.VectorSubcoreMesh(
        core_axis_name="c", subcore_axis_name="s",
        num_cores=_NC, num_subcores=_NS,
    )

    @functools.partial(
        pl.kernel,
        out_type=jax.ShapeDtypeStruct((rows, cols), jnp.float32),
        mesh=mesh,
        scratch_types=[
            pltpu.VMEM((_SC_NBUF, _SC_CHROWS, cols), jnp.float32),
            pltpu.SemaphoreType.DMA((_SC_NBUF,)),
            pltpu.SemaphoreType.DMA((_SC_NBUF,)),
        ],
    )
    def body(x_hbm, out_hbm, buf, in_sems, out_sems):
        wid = lax.axis_index("s") * _NC + lax.axis_index("c")
        base = wid * rows_per_w

        def _in(k, b):
            return pltpu.async_copy(
                x_hbm.at[pl.ds(base + k * _SC_CHROWS, _SC_CHROWS), :],
                buf.at[b], in_sems.at[b],
            )

        def _out(k, b):
            return pltpu.async_copy(
                buf.at[b],
                out_hbm.at[pl.ds(base + k * _SC_CHROWS, _SC_CHROWS), :],
                out_sems.at[b],
            )

        h_in = {}
        h_out = {}
        h_in[0] = _in(0, 0)
        if nchunks > 1:
            h_in[1] = _in(1, 1)
        for k in range(nchunks):
            if k + 2 < nchunks:
                b2 = (k + 2) % _SC_NBUF
                if k - 2 >= 0:
                    h_out[k - 2].wait()
                h_in[k + 2] = _in(k + 2, b2)
            h_in[k].wait()
            h_out[k] = _out(k, k % _SC_NBUF)
        for k in range(max(0, nchunks - 4), nchunks):
            h_out[k].wait()

    return body


def _tc_body(x_ref, q_ref, ms_ref, inv_ref):
    p = pl.program_id(0)
    j = pl.program_id(1)

    @pl.when(p == 0)
    def _phase_minmax():
        bmn = jnp.min(x_ref[...])
        bmx = jnp.max(x_ref[...])

        @pl.when(j == 0)
        def _init():
            ms_ref[0] = bmn
            ms_ref[1] = bmx

        @pl.when(j != 0)
        def _acc():
            ms_ref[0] = jnp.minimum(ms_ref[0], bmn)
            ms_ref[1] = jnp.maximum(ms_ref[1], bmx)

    @pl.when(p == 1)
    def _phase_quant():
        @pl.when(j == 0)
        def _finalize():
            step = (ms_ref[1] - ms_ref[0]) / _LEVELS
            ms_ref[1] = step
            inv_ref[0] = 1.0 / step

        q_ref[...] = jnp.round(
            (x_ref[...] - ms_ref[0]) * inv_ref[0] - _HALF
        ).astype(jnp.int8)


def kernel(input):
    shape = input.shape
    C = shape[-1]
    R = 1
    for s in shape[:-1]:
        R *= s
    x = input.reshape(R, C)

    xc = _make_sc_copy(R, C)(x)

    nb = 16
    bs = R // nb
    q, ms = pl.pallas_call(
        _tc_body,
        grid=(2, nb),
        in_specs=[pl.BlockSpec((bs, C), lambda p, j: (j, 0))],
        out_specs=[
            pl.BlockSpec((bs, C), lambda p, j: (jnp.where(p == 0, 0, j), 0)),
            pl.BlockSpec(memory_space=pltpu.SMEM),
        ],
        out_shape=[
            jax.ShapeDtypeStruct((R, C), jnp.int8),
            jax.ShapeDtypeStruct((2,), jnp.float32),
        ],
        scratch_shapes=[pltpu.SMEM((1,), jnp.float32)],
        compiler_params=pltpu.CompilerParams(
            dimension_semantics=("arbitrary", "arbitrary"),
        ),
    )(x)

    return (xc.reshape(shape), q.reshape(shape), ms)


# speedup vs baseline: 30.9694x; 1.4298x over previous
import jax
import jax.numpy as jnp
from jax.experimental import pallas as pl
from jax.experimental.pallas import tpu as pltpu

_BITS = 8
_LEVELS = float(2 ** _BITS - 1)
_HALF = float(2 ** (_BITS - 1))

_NB = 16
_K = 2


def _body(x_ref, q_ref, xc_ref, ms_ref, keep_ref, inv_ref):
    p = pl.program_id(0)
    j = pl.program_id(1)

    @pl.when(p == 0)
    def _phase_minmax():
        bmn = jnp.min(x_ref[...])
        bmx = jnp.max(x_ref[...])

        @pl.when(j == 0)
        def _init():
            ms_ref[0] = bmn
            ms_ref[1] = bmx

        @pl.when(j != 0)
        def _acc():
            ms_ref[0] = jnp.minimum(ms_ref[0], bmn)
            ms_ref[1] = jnp.maximum(ms_ref[1], bmx)

        for kk in range(_K):
            @pl.when(j == kk + 1)
            def _stash(kk=kk):
                keep_ref[kk] = x_ref[...]

    @pl.when(p == 1)
    def _phase_quant():
        @pl.when(j == 0)
        def _finalize():
            step = (ms_ref[1] - ms_ref[0]) / _LEVELS
            ms_ref[1] = step
            inv_ref[0] = 1.0 / step

        def _emit(x):
            q_ref[...] = jnp.round(
                (x - ms_ref[0]) * inv_ref[0] - _HALF
            ).astype(jnp.int8)
            xc_ref[...] = x

        @pl.when((j == 0) | (j > _K))
        def _from_hbm():
            _emit(x_ref[...])

        for kk in range(_K):
            @pl.when(j == kk + 1)
            def _from_keep(kk=kk):
                _emit(keep_ref[kk])


def kernel(input):
    shape = input.shape
    C = shape[-1]
    R = 1
    for s in shape[:-1]:
        R *= s
    x = input.reshape(R, C)

    nb = _NB
    bs = R // nb

    def _in_map(p, j):
        return (jnp.where((p == 1) & (j <= _K), 0, j), 0)

    q, xc, ms = pl.pallas_call(
        _body,
        grid=(2, nb),
        in_specs=[pl.BlockSpec((bs, C), _in_map)],
        out_specs=[
            pl.BlockSpec((bs, C), lambda p, j: (jnp.where(p == 0, 0, j), 0)),
            pl.BlockSpec((bs, C), lambda p, j: (jnp.where(p == 0, 0, j), 0)),
            pl.BlockSpec(memory_space=pltpu.SMEM),
        ],
        out_shape=[
            jax.ShapeDtypeStruct((R, C), jnp.int8),
            jax.ShapeDtypeStruct((R, C), jnp.float32),
            jax.ShapeDtypeStruct((2,), jnp.float32),
        ],
        scratch_shapes=[
            pltpu.VMEM((_K, bs, C), jnp.float32),
            pltpu.SMEM((1,), jnp.float32),
        ],
        compiler_params=pltpu.CompilerParams(
            dimension_semantics=("arbitrary", "arbitrary"),
        ),
    )(x)

    return (xc.reshape(shape), q.reshape(shape), ms)
